# Initial kernel scaffold; baseline (speedup 1.0000x reference)
import jax
import jax.numpy as jnp
from jax.experimental import pallas as pl


def kernel(full_board_vector, piece_ids, proj_w, proj_b):
    raise NotImplementedError("write your pallas kernel here")


# speedup vs baseline: 13.6788x; 13.6788x over previous
import functools

import jax
import jax.numpy as jnp
from jax import lax
from jax.experimental import pallas as pl
from jax.experimental.pallas import tpu as pltpu
from jax.experimental.pallas import tpu_sc as plsc

B = 16384
C = 11
HW = 64
P = 32
OUT = 64
L = 16
NC = 2
NS = 16
NW = NC * NS
CB = 16
NCHUNK = B // (NW * CB)


def _sc_extract(ids_hbm, board_hbm, raw_hbm, idsb, boardb, fidxb, outb):
    cid = lax.axis_index("c")
    sid = lax.axis_index("s")
    wid = sid * NC + cid
    lane = lax.iota(jnp.int32, L)
    lane32 = lane * P
    zero16 = jnp.zeros((L,), jnp.float32)

    def zrow(r, _):
        plsc.store_scatter(outb, [jnp.full((L,), r, jnp.int32), lane], zero16)
        return 0
    lax.fori_loop(0, CB * P, zrow, 0)

    def chunk(g, _):
        base = (wid * NCHUNK + g) * CB
        pltpu.sync_copy(ids_hbm.at[pl.ds(base, CB)], idsb)
        pltpu.sync_copy(board_hbm.at[pl.ds(base, CB)], boardb)

        sent = jnp.full((L,), HW, jnp.int32)

        def initp(p, _):
            plsc.store_scatter(fidxb, [jnp.full((L,), p, jnp.int32), lane], sent)
            return 0
        lax.fori_loop(0, P + 1, initp, 0)

        def hwstep(i, _):
            hw = (HW - 1) - i
            hwv = jnp.full((L,), 0, jnp.int32) + hw
            idv = plsc.load_gather(idsb, [lane, hwv])
            plsc.store_scatter(fidxb, [idv, lane], hwv)
            return 0
        lax.fori_loop(0, HW, hwstep, 0)

        def pstep(p, _):
            fv = plsc.load_gather(fidxb, [jnp.full((L,), 1, jnp.int32) + p, lane])
            hasf = jnp.where(fv < HW, 1.0, 0.0).astype(jnp.float32)
            fcl = jnp.minimum(fv, HW - 1)
            rowv = lane32 + p
            for c in range(C):
                gv = plsc.load_gather(boardb, [lane, fcl + c * HW])
                plsc.store_scatter(outb, [rowv, jnp.full((L,), c, jnp.int32)],
                                   gv * hasf)
            return 0
        lax.fori_loop(0, P, pstep, 0)

        pltpu.sync_copy(outb, raw_hbm.at[pl.ds(base * P, CB * P)])
        return 0

    lax.fori_loop(0, NCHUNK, chunk, 0)


def _tc_project(x_ref, w_ref, b_ref, o_ref):
    o_ref[...] = jnp.dot(x_ref[...], w_ref[...],
                         preferred_element_type=jnp.float32) + b_ref[...]


def kernel(full_board_vector, piece_ids, proj_w, proj_b):
    ids_flat = piece_ids.reshape(B, HW)
    board_flat = full_board_vector.reshape(B, C * HW)

    sc_call = pl.kernel(
        _sc_extract,
        out_type=jax.ShapeDtypeStruct((B * P, L), jnp.float32),
        mesh=plsc.VectorSubcoreMesh(core_axis_name="c", subcore_axis_name="s",
                                    num_cores=NC, num_subcores=NS),
        compiler_params=pltpu.CompilerParams(needs_layout_passes=False),
        scratch_types=[
            pltpu.VMEM((CB, HW), jnp.int32),
            pltpu.VMEM((CB, C * HW), jnp.float32),
            pltpu.VMEM((P + 1, L), jnp.int32),
            pltpu.VMEM((CB * P, L), jnp.float32),
        ],
    )
    raw = sc_call(ids_flat, board_flat)

    xw = raw.reshape(B * 4, 128)
    wpad = jnp.zeros((L, OUT), jnp.float32).at[:C].set(proj_w.T)
    wblk = jnp.kron(jnp.eye(8, dtype=jnp.float32), wpad)
    bias = jnp.tile(proj_b, 8).reshape(1, 8 * OUT)

    RB = 1024
    rows = B * 4
    out = pl.pallas_call(
        _tc_project,
        grid=(rows // RB,),
        in_specs=[
            pl.BlockSpec((RB, 128), lambda i: (i, 0)),
            pl.BlockSpec((128, 8 * OUT), lambda i: (0, 0)),
            pl.BlockSpec((1, 8 * OUT), lambda i: (0, 0)),
        ],
        out_specs=pl.BlockSpec((RB, 8 * OUT), lambda i: (i, 0)),
        out_shape=jax.ShapeDtypeStruct((rows, 8 * OUT), jnp.float32),
    )(xw, wblk, bias)
    return out.reshape(B, P, OUT)


# speedup vs baseline: 18.0955x; 1.3229x over previous
import functools

import jax
import jax.numpy as jnp
from jax import lax
from jax.experimental import pallas as pl
from jax.experimental.pallas import tpu as pltpu
from jax.experimental.pallas import tpu_sc as plsc

B = 16384
C = 11
HW = 64
P = 32
OUT = 64
L = 16
NC = 2
NS = 16
NW = NC * NS
CB = 16
NCHUNK = B // (NW * CB)


def _sc_extract(ids_hbm, board_hbm, raw_hbm, idsb, boardb, fidxb, outb):
    cid = lax.axis_index("c")
    sid = lax.axis_index("s")
    wid = sid * NC + cid
    lane = lax.iota(jnp.int32, L)
    lane4 = lane * 4
    zero16 = jnp.zeros((L,), jnp.float32)
    sent = jnp.full((L,), HW, jnp.int32)

    for r in range(CB * 4):
        for gg in range(8):
            outb[r, pl.ds(gg * L, L)] = zero16

    def chunk(g, _):
        base = (wid * NCHUNK + g) * CB
        pltpu.sync_copy(ids_hbm.at[pl.ds(base, CB)], idsb)
        pltpu.sync_copy(board_hbm.at[pl.ds(base, CB)], boardb)

        for p in range(P + 1):
            fidxb[p] = sent

        for i in range(HW):
            hw = (HW - 1) - i
            hwv = jnp.full((L,), hw, jnp.int32)
            idv = plsc.load_gather(idsb, [lane, hwv])
            plsc.store_scatter(fidxb, [idv, lane], hwv)

        for p in range(P):
            fv = fidxb[p + 1]
            hasf = jnp.where(fv < HW, 1.0, 0.0).astype(jnp.float32)
            fcl = jnp.minimum(fv, HW - 1)
            rowv = lane4 + (p // 8)
            for c in range(C):
                gv = plsc.load_gather(boardb, [lane, fcl + c * HW])
                plsc.store_scatter(
                    outb, [rowv, jnp.full((L,), (p % 8) * L + c, jnp.int32)],
                    gv * hasf)

        pltpu.sync_copy(outb, raw_hbm.at[pl.ds(base * 4, CB * 4)])
        return 0

    lax.fori_loop(0, NCHUNK, chunk, 0)


def _tc_project(x_ref, w_ref, b_ref, o_ref):
    o_ref[...] = jnp.dot(x_ref[...], w_ref[...],
                         preferred_element_type=jnp.float32) + b_ref[...]


def kernel(full_board_vector, piece_ids, proj_w, proj_b):
    ids_flat = piece_ids.reshape(B, HW)
    board_flat = full_board_vector.reshape(B, C * HW)

    sc_call = pl.kernel(
        _sc_extract,
        out_type=jax.ShapeDtypeStruct((B * 4, 128), jnp.float32),
        mesh=plsc.VectorSubcoreMesh(core_axis_name="c", subcore_axis_name="s",
                                    num_cores=NC, num_subcores=NS),
        compiler_params=pltpu.CompilerParams(needs_layout_passes=False),
        scratch_types=[
            pltpu.VMEM((CB, HW), jnp.int32),
            pltpu.VMEM((CB, C * HW), jnp.float32),
            pltpu.VMEM((P + 1, L), jnp.int32),
            pltpu.VMEM((CB * 4, 128), jnp.float32),
        ],
    )
    xw = sc_call(ids_flat, board_flat)
    wpad = jnp.zeros((L, OUT), jnp.float32).at[:C].set(proj_w.T)
    wblk = jnp.kron(jnp.eye(8, dtype=jnp.float32), wpad)
    bias = jnp.tile(proj_b, 8).reshape(1, 8 * OUT)

    RB = 1024
    rows = B * 4
    out = pl.pallas_call(
        _tc_project,
        grid=(rows // RB,),
        in_specs=[
            pl.BlockSpec((RB, 128), lambda i: (i, 0)),
            pl.BlockSpec((128, 8 * OUT), lambda i: (0, 0)),
            pl.BlockSpec((1, 8 * OUT), lambda i: (0, 0)),
        ],
        out_specs=pl.BlockSpec((RB, 8 * OUT), lambda i: (i, 0)),
        out_shape=jax.ShapeDtypeStruct((rows, 8 * OUT), jnp.float32),
    )(xw, wblk, bias)
    return out.reshape(B, P, OUT)


# speedup vs baseline: 49.1731x; 2.7174x over previous
import jax
import jax.numpy as jnp
from jax import lax
from jax.experimental import pallas as pl
from jax.experimental.pallas import tpu as pltpu
from jax.experimental.pallas import tpu_sc as plsc

B = 16384
C = 11
HW = 64
P = 32
OUT = 64
L = 16
NC = 2
NS = 16
NW = NC * NS
WB = B // NW
NH = WB // L


def _sc_extract(ids_hbm, board_hbm, raw_hbm, idsb, boardb, fidxb, outb):
    cid = lax.axis_index("c")
    sid = lax.axis_index("s")
    wid = sid * NC + cid
    col0 = wid * WB
    lane = lax.iota(jnp.int32, L)
    zero16 = jnp.zeros((L,), jnp.float32)
    sent = jnp.full((L,), HW, jnp.int32)

    for h in range(NH):
        boardb[HW, pl.ds(h * L, L)] = zero16

    def initp(p, _):
        def inith(h, _):
            plsc.store_scatter(fidxb, [jnp.full((L,), 0, jnp.int32) + p,
                                       lane + h * L], sent)
            return 0
        return lax.fori_loop(0, NH, inith, 0)
    lax.fori_loop(0, P + 1, initp, 0)

    pltpu.sync_copy(ids_hbm.at[:, pl.ds(col0, WB)], idsb)

    def hwstep(i, _):
        hw = (HW - 1) - i
        hwv = jnp.full((L,), 0, jnp.int32) + hw
        for h in range(NH):
            lv = lane + h * L
            idv = plsc.load_gather(idsb, [hwv, lv])
            plsc.store_scatter(fidxb, [idv, lv], hwv)
        return 0
    lax.fori_loop(0, HW, hwstep, 0)

    for c in range(C):
        pltpu.sync_copy(board_hbm.at[pl.ds(c * HW, HW), pl.ds(col0, WB)],
                        boardb.at[pl.ds(0, HW)])

        def pstep(p, _):
            pv = jnp.full((L,), 1, jnp.int32) + p
            for h in range(NH):
                lv = lane + h * L
                fv = plsc.load_gather(fidxb, [pv, lv])
                gv = plsc.load_gather(boardb, [fv, lv])
                plsc.store_scatter(outb, [pv - 1, jnp.full((L,), 0, jnp.int32),
                                          lv], gv)
            return 0
        lax.fori_loop(0, P, pstep, 0)

        pltpu.sync_copy(outb, raw_hbm.at[:, pl.ds(c, 1), pl.ds(col0, WB)])

    def fillh(h, val):
        for p in range(P):
            plsc.store_scatter(outb, [jnp.full((L,), p, jnp.int32),
                                      jnp.full((L,), 0, jnp.int32),
                                      lane + h * L], val)
        return val
    lax.fori_loop(0, NH, fillh, zero16)
    for c in range(C, L - 1):
        pltpu.sync_copy(outb, raw_hbm.at[:, pl.ds(c, 1), pl.ds(col0, WB)])
    lax.fori_loop(0, NH, fillh, zero16 + 1.0)
    pltpu.sync_copy(outb, raw_hbm.at[:, pl.ds(L - 1, 1), pl.ds(col0, WB)])


def _tc_project(w_ref, x_ref, o_ref):
    o_ref[...] = jnp.dot(w_ref[...], x_ref[...],
                         preferred_element_type=jnp.float32)


def kernel(full_board_vector, piece_ids, proj_w, proj_b):
    ids_t = piece_ids.transpose(1, 2, 0).reshape(HW, B)
    board_t = full_board_vector.transpose(1, 2, 3, 0).reshape(C * HW, B)

    sc_call = pl.kernel(
        _sc_extract,
        out_type=jax.ShapeDtypeStruct((P, L, B), jnp.float32),
        mesh=plsc.VectorSubcoreMesh(core_axis_name="c", subcore_axis_name="s",
                                    num_cores=NC, num_subcores=NS),
        compiler_params=pltpu.CompilerParams(needs_layout_passes=False),
        scratch_types=[
            pltpu.VMEM((HW, WB), jnp.int32),
            pltpu.VMEM((HW + 1, WB), jnp.float32),
            pltpu.VMEM((P + 1, WB), jnp.int32),
            pltpu.VMEM((P, 1, WB), jnp.float32),
        ],
    )
    raw_t = sc_call(ids_t, board_t).reshape(P * L, B)

    wb = jnp.zeros((OUT, L), jnp.float32).at[:, :C].set(proj_w)
    wb = wb.at[:, L - 1].set(proj_b)
    w16 = jnp.kron(jnp.eye(L, dtype=jnp.float32), wb)

    NB = 2048
    y = pl.pallas_call(
        _tc_project,
        grid=(2, B // NB),
        in_specs=[
            pl.BlockSpec((L * OUT, L * L), lambda g, nb: (0, 0)),
            pl.BlockSpec((L * L, NB), lambda g, nb: (g, nb)),
        ],
        out_specs=pl.BlockSpec((L * OUT, NB), lambda g, nb: (g, nb)),
        out_shape=jax.ShapeDtypeStruct((P * OUT, B), jnp.float32),
    )(w16, raw_t)
    return y.reshape(P, OUT, B).transpose(2, 0, 1)


# speedup vs baseline: 91.0878x; 1.8524x over previous
import jax
import jax.numpy as jnp
from jax import lax
from jax.experimental import pallas as pl
from jax.experimental.pallas import tpu as pltpu
from jax.experimental.pallas import tpu_sc as plsc

B = 16384
C = 11
HW = 64
P = 32
OUT = 64
L = 16
NC = 2
NS = 16
NW = NC * NS
WB = B // NW
NH = WB // L


def _sc_extract(ids_hbm, board_hbm, raw_hbm, idsb, boardb, fidxb, outb):
    cid = lax.axis_index("c")
    sid = lax.axis_index("s")
    wid = sid * NC + cid
    col0 = wid * WB
    lane = lax.iota(jnp.int32, L)
    zero16 = jnp.zeros((L,), jnp.float32)
    sent = jnp.full((L,), HW, jnp.int32)

    @plsc.parallel_loop(0, NH)
    def _init(h):
        hh = h * L
        boardb[HW, pl.ds(hh, L)] = zero16
        for p in range(P + 1):
            fidxb[p, pl.ds(hh, L)] = sent

    pltpu.sync_copy(ids_hbm.at[:, pl.ds(col0, WB)], idsb)

    @plsc.parallel_loop(0, NH)
    def _scan(h):
        hh = h * L
        lv = lane + hh
        for hw in range(HW - 1, -1, -1):
            idv = idsb[hw, pl.ds(hh, L)]
            plsc.store_scatter(fidxb, [idv, lv], jnp.full((L,), hw, jnp.int32))

    def cpass(c, _):
        pltpu.sync_copy(board_hbm.at[pl.ds(c * HW, HW), pl.ds(col0, WB)],
                        boardb.at[pl.ds(0, HW)])

        @plsc.parallel_loop(0, NH)
        def _gather(h):
            hh = h * L
            lv = lane + hh
            for p in range(P):
                fv = fidxb[p + 1, pl.ds(hh, L)]
                gv = plsc.load_gather(boardb, [fv, lv])
                outb[p, 0, pl.ds(hh, L)] = gv

        pltpu.sync_copy(outb, raw_hbm.at[:, pl.ds(c, 1), pl.ds(col0, WB)])
        return 0
    lax.fori_loop(0, C, cpass, 0)

    @plsc.parallel_loop(0, NH)
    def _fill0(h):
        for p in range(P):
            outb[p, 0, pl.ds(h * L, L)] = zero16

    for c in range(C, L - 1):
        pltpu.sync_copy(outb, raw_hbm.at[:, pl.ds(c, 1), pl.ds(col0, WB)])

    @plsc.parallel_loop(0, NH)
    def _fill1(h):
        for p in range(P):
            outb[p, 0, pl.ds(h * L, L)] = zero16 + 1.0

    pltpu.sync_copy(outb, raw_hbm.at[:, pl.ds(L - 1, 1), pl.ds(col0, WB)])


def _tc_project(w_ref, x_ref, o_ref):
    o_ref[...] = jnp.dot(w_ref[...], x_ref[...],
                         preferred_element_type=jnp.float32)


def kernel(full_board_vector, piece_ids, proj_w, proj_b):
    ids_t = piece_ids.transpose(1, 2, 0).reshape(HW, B)
    board_t = full_board_vector.transpose(1, 2, 3, 0).reshape(C * HW, B)

    sc_call = pl.kernel(
        _sc_extract,
        out_type=jax.ShapeDtypeStruct((P, L, B), jnp.float32),
        mesh=plsc.VectorSubcoreMesh(core_axis_name="c", subcore_axis_name="s",
                                    num_cores=NC, num_subcores=NS),
        compiler_params=pltpu.CompilerParams(needs_layout_passes=False),
        scratch_types=[
            pltpu.VMEM((HW, WB), jnp.int32),
            pltpu.VMEM((HW + 1, WB), jnp.float32),
            pltpu.VMEM((P + 1, WB), jnp.int32),
            pltpu.VMEM((P, 1, WB), jnp.float32),
        ],
    )
    raw_t = sc_call(ids_t, board_t).reshape(P * L, B)

    wb = jnp.zeros((OUT, L), jnp.float32).at[:, :C].set(proj_w)
    wb = wb.at[:, L - 1].set(proj_b)
    w16 = jnp.kron(jnp.eye(L, dtype=jnp.float32), wb)

    NB = 2048
    y = pl.pallas_call(
        _tc_project,
        grid=(2, B // NB),
        in_specs=[
            pl.BlockSpec((L * OUT, L * L), lambda g, nb: (0, 0)),
            pl.BlockSpec((L * L, NB), lambda g, nb: (g, nb)),
        ],
        out_specs=pl.BlockSpec((L * OUT, NB), lambda g, nb: (g, nb)),
        out_shape=jax.ShapeDtypeStruct((P * OUT, B), jnp.float32),
    )(w16, raw_t)
    return y.reshape(P, OUT, B).transpose(2, 0, 1)


# speedup vs baseline: 102.2697x; 1.1228x over previous
import jax
import jax.numpy as jnp
from jax import lax
from jax.experimental import pallas as pl
from jax.experimental.pallas import tpu as pltpu
from jax.experimental.pallas import tpu_sc as plsc

B = 16384
C = 11
HW = 64
P = 32
OUT = 64
L = 16
NC = 2
NS = 16
NW = NC * NS
WB = B // NW
NH = WB // L


def _sc_extract(ids_hbm, board_hbm, raw_hbm,
                bb0, bb1, fidxb, ob0, ob1, si0, si1, so0, so1):
    cid = lax.axis_index("c")
    sid = lax.axis_index("s")
    wid = sid * NC + cid
    col0 = wid * WB
    lane = lax.iota(jnp.int32, L)
    zero16 = jnp.zeros((L,), jnp.float32)
    sent = jnp.full((L,), HW, jnp.int32)

    bbs = (bb0, bb1)
    obs = (ob0, ob1)
    sis = (si0, si1)
    sos = (so0, so1)

    def in_dma(c, buf):
        return pltpu.make_async_copy(
            board_hbm.at[pl.ds(c * HW, HW), pl.ds(col0, WB)],
            bbs[buf].at[pl.ds(0, HW)], sis[buf])

    def out_dma(c, buf):
        return pltpu.make_async_copy(
            obs[buf], raw_hbm.at[:, pl.ds(c, 1), pl.ds(col0, WB)], sos[buf])

    @plsc.parallel_loop(0, NH)
    def _init(h):
        hh = h * L
        bb0[HW, pl.ds(hh, L)] = zero16
        bb1[HW, pl.ds(hh, L)] = zero16
        for p in range(P + 1):
            fidxb[p, pl.ds(hh, L)] = sent

    pltpu.sync_copy(ids_hbm.at[:, pl.ds(col0, WB)], bb0.at[pl.ds(0, HW)])
    in_dma(0, 1).start()

    @plsc.parallel_loop(0, NH)
    def _scan(h):
        hh = h * L
        lv = lane + hh
        for hw in range(HW - 1, -1, -1):
            idv = plsc.bitcast(bb0[hw, pl.ds(hh, L)], jnp.int32)
            plsc.store_scatter(fidxb, [idv, lv], jnp.full((L,), hw, jnp.int32))

    def gather_pass(buf, obuf):
        src = bbs[buf]
        dst = obs[obuf]

        @plsc.parallel_loop(0, NH)
        def _gather(h):
            hh = h * L
            lv = lane + hh
            for p in range(P):
                fv = fidxb[p + 1, pl.ds(hh, L)]
                gv = plsc.load_gather(src, [fv, lv])
                dst[p, 0, pl.ds(hh, L)] = gv

    def pair(k, _):
        c0 = 2 * k
        in_dma(c0, 1).wait()

        @pl.when(k > 0)
        def _():
            out_dma(c0 - 2, 0).wait()
        in_dma(c0 + 1, 0).start()
        gather_pass(1, 0)
        out_dma(c0, 0).start()

        c1 = 2 * k + 1
        in_dma(c1, 0).wait()

        @pl.when(k > 0)
        def _():
            out_dma(c1 - 2, 1).wait()
        in_dma(c1 + 1, 1).start()
        gather_pass(0, 1)
        out_dma(c1, 1).start()
        return 0
    lax.fori_loop(0, (C - 1) // 2, pair, 0)

    in_dma(C - 1, 1).wait()
    out_dma(C - 3, 0).wait()
    gather_pass(1, 0)
    out_dma(C - 1, 0).start()

    out_dma(C - 2, 1).wait()

    @plsc.parallel_loop(0, NH)
    def _fill0(h):
        for p in range(P):
            ob1[p, 0, pl.ds(h * L, L)] = zero16

    for c in range(C, L - 1):
        pltpu.sync_copy(ob1, raw_hbm.at[:, pl.ds(c, 1), pl.ds(col0, WB)])

    out_dma(C - 1, 0).wait()

    @plsc.parallel_loop(0, NH)
    def _fill1(h):
        for p in range(P):
            ob0[p, 0, pl.ds(h * L, L)] = zero16 + 1.0

    pltpu.sync_copy(ob0, raw_hbm.at[:, pl.ds(L - 1, 1), pl.ds(col0, WB)])


def _tc_project(w_ref, x_ref, o_ref):
    o_ref[...] = jnp.dot(w_ref[...], x_ref[...],
                         preferred_element_type=jnp.float32)


def kernel(full_board_vector, piece_ids, proj_w, proj_b):
    ids_t = jax.lax.bitcast_convert_type(
        piece_ids.transpose(1, 2, 0).reshape(HW, B), jnp.float32)
    board_t = full_board_vector.transpose(1, 2, 3, 0).reshape(C * HW, B)

    sc_call = pl.kernel(
        _sc_extract,
        out_type=jax.ShapeDtypeStruct((P, L, B), jnp.float32),
        mesh=plsc.VectorSubcoreMesh(core_axis_name="c", subcore_axis_name="s",
                                    num_cores=NC, num_subcores=NS),
        compiler_params=pltpu.CompilerParams(needs_layout_passes=False),
        scratch_types=[
            pltpu.VMEM((HW + 1, WB), jnp.float32),
            pltpu.VMEM((HW + 1, WB), jnp.float32),
            pltpu.VMEM((P + 1, WB), jnp.int32),
            pltpu.VMEM((P, 1, WB), jnp.float32),
            pltpu.VMEM((P, 1, WB), jnp.float32),
            pltpu.SemaphoreType.DMA,
            pltpu.SemaphoreType.DMA,
            pltpu.SemaphoreType.DMA,
            pltpu.SemaphoreType.DMA,
        ],
    )
    raw_t = sc_call(ids_t, board_t).reshape(P * L, B)

    wb = jnp.zeros((OUT, L), jnp.float32).at[:, :C].set(proj_w)
    wb = wb.at[:, L - 1].set(proj_b)
    w16 = jnp.kron(jnp.eye(L, dtype=jnp.float32), wb)

    NB = 2048
    y = pl.pallas_call(
        _tc_project,
        grid=(2, B // NB),
        in_specs=[
            pl.BlockSpec((L * OUT, L * L), lambda g, nb: (0, 0)),
            pl.BlockSpec((L * L, NB), lambda g, nb: (g, nb)),
        ],
        out_specs=pl.BlockSpec((L * OUT, NB), lambda g, nb: (g, nb)),
        out_shape=jax.ShapeDtypeStruct((P * OUT, B), jnp.float32),
    )(w16, raw_t)
    return y.reshape(P, OUT, B).transpose(2, 0, 1)
